# baseline (speedup 1.0000x reference)
import jax
import jax.numpy as jnp
from jax.experimental import pallas as pl
from jax.experimental.pallas import tpu as pltpu


_BLK_S = 256


def _add_kernel(x_ref, pos_ref, o_ref):
    o_ref[...] = x_ref[...] + pos_ref[...][None]


def kernel(x, pos_table):
    batch, seq_len, dim = x.shape
    blk = _BLK_S
    grid = (seq_len // blk,)
    return pl.pallas_call(
        _add_kernel,
        grid=grid,
        in_specs=[
            pl.BlockSpec((batch, blk, dim), lambda s: (0, s, 0)),
            pl.BlockSpec((blk, dim), lambda s: (s, 0)),
        ],
        out_specs=pl.BlockSpec((batch, blk, dim), lambda s: (0, s, 0)),
        out_shape=jax.ShapeDtypeStruct((batch, seq_len, dim), x.dtype),
        compiler_params=pltpu.CompilerParams(
            dimension_semantics=("arbitrary",),
        ),
    )(x, pos_table)


# speedup vs baseline: 1.0047x; 1.0047x over previous
import jax
import jax.numpy as jnp
from jax.experimental import pallas as pl


_BLK_S = 1024


def _add_kernel(x_ref, pos_ref, o_ref):
    o_ref[...] = x_ref[...] + pos_ref[...]


def kernel(x, pos_table):
    batch, seq_len, dim = x.shape
    blk = _BLK_S
    grid = (seq_len // blk, batch)
    return pl.pallas_call(
        _add_kernel,
        grid=grid,
        in_specs=[
            pl.BlockSpec((1, blk, dim), lambda s, b: (b, s, 0)),
            pl.BlockSpec((blk, dim), lambda s, b: (s, 0)),
        ],
        out_specs=pl.BlockSpec((1, blk, dim), lambda s, b: (b, s, 0)),
        out_shape=jax.ShapeDtypeStruct((batch, seq_len, dim), x.dtype),
    )(x, pos_table)


# speedup vs baseline: 1.0057x; 1.0010x over previous
import jax
import jax.numpy as jnp
from jax.experimental import pallas as pl
from jax.experimental.pallas import tpu as pltpu


_BLK_S = 1024


def _add_kernel(x_ref, pos_ref, o_ref):
    o_ref[...] = x_ref[...] + pos_ref[...]


def kernel(x, pos_table):
    batch, seq_len, dim = x.shape
    blk = _BLK_S
    grid = (seq_len // blk, batch)
    return pl.pallas_call(
        _add_kernel,
        grid=grid,
        in_specs=[
            pl.BlockSpec((1, blk, dim), lambda s, b: (b, s, 0)),
            pl.BlockSpec((blk, dim), lambda s, b: (s, 0)),
        ],
        out_specs=pl.BlockSpec((1, blk, dim), lambda s, b: (b, s, 0)),
        out_shape=jax.ShapeDtypeStruct((batch, seq_len, dim), x.dtype),
        compiler_params=pltpu.CompilerParams(
            dimension_semantics=("parallel", "parallel"),
        ),
    )(x, pos_table)


# speedup vs baseline: 1.0069x; 1.0012x over previous
import jax
import jax.numpy as jnp
from jax.experimental import pallas as pl


_BLK_S = 1024


def _add_kernel(x_ref, pos_ref, o_ref):
    o_ref[...] = x_ref[...] + pos_ref[...]


def kernel(x, pos_table):
    batch, seq_len, dim = x.shape
    blk = _BLK_S
    grid = (seq_len // blk, batch)
    return pl.pallas_call(
        _add_kernel,
        grid=grid,
        in_specs=[
            pl.BlockSpec((1, blk, dim), lambda s, b: (b, s, 0)),
            pl.BlockSpec((blk, dim), lambda s, b: (s, 0)),
        ],
        out_specs=pl.BlockSpec((1, blk, dim), lambda s, b: (b, s, 0)),
        out_shape=jax.ShapeDtypeStruct((batch, seq_len, dim), x.dtype),
    )(x, pos_table)
